# baseline (speedup 1.0000x reference)
import functools

import jax
import jax.numpy as jnp
from jax import lax
from jax.experimental import pallas as pl
from jax.experimental.pallas import tpu as pltpu
from jax.experimental.pallas import tpu_sc as plsc

EMB = 64
N = 4096 * 200
NC, NS = 2, 16
NW = NC * NS
PER_W = N // NW
CH = 1024
NCHUNK = PER_W // CH
IPG = 128
GPC = CH // IPG


def _zindex16(v):
    x = v & 0x3FF
    y = lax.shift_right_logical(v, 10)

    def spread(b):
        b = (b | (b << 8)) & 16711935
        b = (b | (b << 4)) & 252645135
        b = (b | (b << 2)) & 858993459
        b = (b | (b << 1)) & 1431655765
        return b

    return (spread(y) << 1) | spread(x)


_MESH = plsc.VectorSubcoreMesh(core_axis_name="c", subcore_axis_name="s")


@functools.partial(
    pl.kernel,
    out_type=jax.ShapeDtypeStruct((N, EMB), jnp.float32),
    mesh=_MESH,
    compiler_params=pltpu.CompilerParams(use_tc_tiling_on_sc=False),
    scratch_types=[
        pltpu.VMEM((CH,), jnp.int32),
        pltpu.VMEM((CH,), jnp.int32),
        pltpu.VMEM((CH, EMB), jnp.float32),
        pltpu.SemaphoreType.DMA,
    ],
)
def _sc_lookup(loc_hbm, table_hbm, out_hbm, ids_v, idx_v, rows_v, sem):
    wid = lax.axis_index("s") * NC + lax.axis_index("c")
    base = wid * PER_W

    def chunk(c, carry):
        off = base + c * CH
        pltpu.sync_copy(loc_hbm.at[pl.ds(off, CH)], ids_v)

        def zstep(i, carry2):
            v = ids_v[pl.ds(i * 16, 16)]
            idx_v[pl.ds(i * 16, 16)] = _zindex16(v)
            return carry2

        lax.fori_loop(0, CH // 16, zstep, 0)

        copies = [
            pltpu.async_copy(
                table_hbm.at[idx_v.at[pl.ds(j * IPG, IPG)]],
                rows_v.at[pl.ds(j * IPG, IPG)],
                sem,
            )
            for j in range(GPC)
        ]
        for cp in copies:
            cp.wait()
        pltpu.sync_copy(rows_v, out_hbm.at[pl.ds(off, CH)])
        return carry

    lax.fori_loop(0, NCHUNK, chunk, 0)


def kernel(location_id, table):
    flat = location_id.reshape(-1)
    out = _sc_lookup(flat, table)
    return out.reshape(location_id.shape + (EMB,))


# speedup vs baseline: 1.0108x; 1.0108x over previous
import functools

import jax
import jax.numpy as jnp
from jax import lax
from jax.experimental import pallas as pl
from jax.experimental.pallas import tpu as pltpu
from jax.experimental.pallas import tpu_sc as plsc

EMB = 64
N = 4096 * 200
NC, NS = 2, 16
NW = NC * NS
PER_W = N // NW
CH = 512
NCHUNK = PER_W // CH
IPG = 128
GPC = CH // IPG
NBUF = 3


def _zindex16(v):
    x = v & 0x3FF
    y = lax.shift_right_logical(v, 10)

    def spread(b):
        b = (b | (b << 8)) & 16711935
        b = (b | (b << 4)) & 252645135
        b = (b | (b << 2)) & 858993459
        b = (b | (b << 1)) & 1431655765
        return b

    return (spread(y) << 1) | spread(x)


_MESH = plsc.VectorSubcoreMesh(core_axis_name="c", subcore_axis_name="s")


@functools.partial(
    pl.kernel,
    out_type=jax.ShapeDtypeStruct((N, EMB), jnp.float32),
    mesh=_MESH,
    compiler_params=pltpu.CompilerParams(use_tc_tiling_on_sc=False),
    scratch_types=[
        pltpu.VMEM((PER_W,), jnp.int32),
        pltpu.VMEM((NBUF, CH, EMB), jnp.float32),
        pltpu.SemaphoreType.DMA,
        pltpu.SemaphoreType.DMA,
        pltpu.SemaphoreType.DMA,
        pltpu.SemaphoreType.DMA,
        pltpu.SemaphoreType.DMA,
        pltpu.SemaphoreType.DMA,
    ],
)
def _sc_lookup(loc_hbm, table_hbm, out_hbm, idx_all, rows, sg0, sg1, sg2,
               so0, so1, so2):
    sem_g = (sg0, sg1, sg2)
    sem_o = (so0, so1, so2)
    wid = lax.axis_index("s") * NC + lax.axis_index("c")
    base = wid * PER_W

    pltpu.sync_copy(loc_hbm.at[pl.ds(base, PER_W)], idx_all)

    def zstep(i, carry):
        sl = pl.ds(i * 16, 16)
        idx_all[sl] = _zindex16(idx_all[sl])
        return carry

    lax.fori_loop(0, PER_W // 16, zstep, 0)

    def fire_gathers(c, b):
        for j in range(GPC):
            pltpu.async_copy(
                table_hbm.at[idx_all.at[pl.ds(c * CH + j * IPG, IPG)]],
                rows.at[b].at[pl.ds(j * IPG, IPG)],
                sem_g[b],
            )

    def wait_gathers(c, b):
        for j in range(GPC):
            pltpu.make_async_copy(
                table_hbm.at[idx_all.at[pl.ds(c * CH + j * IPG, IPG)]],
                rows.at[b].at[pl.ds(j * IPG, IPG)],
                sem_g[b],
            ).wait()

    def fire_out(c, b):
        pltpu.async_copy(rows.at[b], out_hbm.at[pl.ds(base + c * CH, CH)],
                         sem_o[b])

    def wait_out(c, b):
        pltpu.make_async_copy(rows.at[b], out_hbm.at[pl.ds(base + c * CH, CH)],
                              sem_o[b]).wait()

    for k in range(NBUF):
        fire_gathers(k, k)

    def step(c, carry):
        @pl.when(jnp.logical_and(c > 0, c + NBUF - 1 < NCHUNK))
        def _refill():
            for b in range(NBUF):

                @pl.when((c - 1) % NBUF == b)
                def _():
                    wait_out(c - 1, b)
                    fire_gathers(c + NBUF - 1, b)

        for b in range(NBUF):

            @pl.when(c % NBUF == b)
            def _drain():
                wait_gathers(c, b)
                fire_out(c, b)

        return carry

    lax.fori_loop(0, NCHUNK, step, 0)

    for k in range(NBUF):
        c = NCHUNK - NBUF + k
        wait_out(c, c % NBUF)


def kernel(location_id, table):
    flat = location_id.reshape(-1)
    out = _sc_lookup(flat, table)
    return out.reshape(location_id.shape + (EMB,))
